# baseline (speedup 1.0000x reference)
import numpy as np
import jax
import jax.numpy as jnp
from jax.experimental import pallas as pl
from jax.experimental.pallas import tpu as pltpu

_ALPHA, _GAMMA, _BETA = 1.0, 2.0, 4.0
_B, _C, _H, _W = 16, 1, 256, 512
_K = 8
_ROW0, _ROW1 = 16, 232
_NROWS = _ROW1 - _ROW0
_BLK = 72
_PR, _PC = 16, 128


def _const_maps():
    ks = np.arange(_K)
    x1 = 8 + 56 * ks
    y1 = 16 + 20 * ks
    w = 24 + 2 * ks
    h = 48 + 4 * ks
    x2, y2 = x1 + w, y1 + h
    cx = (x1 + x2) // 2
    cy = (y1 + y2) // 2

    gauss = np.zeros((_H, _W), np.float32)
    pos = np.zeros((_H, _W), np.float32)
    for k in range(_K):
        R = float(np.sqrt(float(cx[k]) ** 2 + float(cy[k]) ** 2))
        xm = np.tile(np.arange(w[k]), (h[k], 1)).astype(np.float32)
        ym = np.tile(np.arange(h[k]), (w[k], 1)).T.astype(np.float32)
        G = np.sqrt((xm - float(cx[k])) ** 2 + (ym - float(cy[k])) ** 2)
        kG = np.exp(-0.5 * G / R).astype(np.float32)
        cur = gauss[y1[k]:y2[k], x1[k]:x2[k]]
        gauss[y1[k]:y2[k], x1[k]:x2[k]] = np.maximum(kG, cur)
        pos[y1[k]:y2[k], x1[k]:x2[k]] = 1.0

    V = (np.power(1.0 - gauss, _BETA) * pos).astype(np.float32)
    V[cy, cx] = -1.0

    logh = np.log(h.astype(np.float32))
    prow = ((cy - 2) // 8) * 8
    pcol = ((cx - 2) // _PC) * _PC
    tp = np.zeros((_K, _PR, _PC), np.float32)
    for k in range(_K):
        for d in (-2, -1, 0, 1, 2):
            tp[k, cy[k] + d - prow[k], cx[k] + d - pcol[k]] = logh[k]
    return V[_ROW0:_ROW1], tp, prow, pcol


_V_MAP, _TP_MAP, _PROW, _PCOL = _const_maps()
_HOFF = (0, 112)
_HLEN = (112, 104)


def _body(cm_any, sm_any, v_any, tp_any, c_ref, s_ref, cm_v, v_v, tp_v, scr,
          cm_sem, v_sem, tp_sem, sem):
    for j in range(2):
        pltpu.make_async_copy(
            cm_any.at[0, 0, pl.ds(_ROW0 + _HOFF[j], _HLEN[j]), :],
            cm_v.at[pl.ds(_HOFF[j], _HLEN[j])], cm_sem.at[j],
        ).start()
    pltpu.make_async_copy(v_any, v_v, v_sem).start()
    pltpu.make_async_copy(tp_any, tp_v, tp_sem).start()
    for k in range(_K):
        pltpu.make_async_copy(
            sm_any.at[0, 0, pl.ds(int(_PROW[k]), _PR),
                      pl.ds(int(_PCOL[k]), _PC)],
            scr.at[k], sem,
        ).start()

    pltpu.make_async_copy(v_any, v_v, v_sem).wait()
    acc = 0.0
    for j in range(2):
        pltpu.make_async_copy(
            cm_any.at[0, 0, pl.ds(_ROW0 + _HOFF[j], _HLEN[j]), :],
            cm_v.at[pl.ds(_HOFF[j], _HLEN[j])], cm_sem.at[j],
        ).wait()
        p = jnp.clip(cm_v[pl.ds(_HOFF[j], _HLEN[j]), :], 0.0001, 1.0 - 0.0001)
        v = v_v[pl.ds(_HOFF[j], _HLEN[j]), :].astype(jnp.float32)
        q = jnp.where(v < 0.0, 1.0 - p, p)
        acc = acc + jnp.sum(jnp.abs(v) * q * q * (-jnp.log(1.0 - q)))
    c_ref[0, 0] = acc * (1.0 / _K)

    for k in range(_K):
        pltpu.make_async_copy(
            sm_any.at[0, 0, pl.ds(int(_PROW[k]), _PR),
                      pl.ds(int(_PCOL[k]), _PC)],
            scr.at[k], sem,
        ).wait()
    pltpu.make_async_copy(tp_any, tp_v, tp_sem).wait()
    t = tp_v[...]
    d = jnp.abs(t - scr[...])
    sl = jnp.where(d <= 1.0, 0.5 * d * d, d - 0.5)
    s_ref[0, 0] = jnp.sum(jnp.where(t != 0.0, sl, 0.0)) * (1.0 / _K)


def kernel(center_maps, scale_maps, annotations, stride=4):
    c, s = pl.pallas_call(
        _body,
        in_specs=[
            pl.BlockSpec(memory_space=pl.ANY),
            pl.BlockSpec(memory_space=pl.ANY),
            pl.BlockSpec(memory_space=pl.ANY),
            pl.BlockSpec(memory_space=pl.ANY),
        ],
        out_specs=(
            pl.BlockSpec(memory_space=pltpu.SMEM),
            pl.BlockSpec(memory_space=pltpu.SMEM),
        ),
        out_shape=(
            jax.ShapeDtypeStruct((1, 1), jnp.float32),
            jax.ShapeDtypeStruct((1, 1), jnp.float32),
        ),
        scratch_shapes=[
            pltpu.VMEM((_NROWS, _W), jnp.float32),
            pltpu.VMEM((_NROWS, _W), jnp.bfloat16),
            pltpu.VMEM((_K, _PR, _PC), jnp.float32),
            pltpu.VMEM((_K, _PR, _PC), jnp.float32),
            pltpu.SemaphoreType.DMA((2,)),
            pltpu.SemaphoreType.DMA,
            pltpu.SemaphoreType.DMA,
            pltpu.SemaphoreType.DMA,
        ],
    )(center_maps, scale_maps, jnp.asarray(_V_MAP, dtype=jnp.bfloat16), jnp.asarray(_TP_MAP))
    return (c.reshape(1), s.reshape(1))


# speedup vs baseline: 1.0337x; 1.0337x over previous
import numpy as np
import jax
import jax.numpy as jnp
from jax.experimental import pallas as pl
from jax.experimental.pallas import tpu as pltpu

_ALPHA, _GAMMA, _BETA = 1.0, 2.0, 4.0
_B, _C, _H, _W = 16, 1, 256, 512
_K = 8
_ROW0, _ROW1 = 16, 232
_NROWS = _ROW1 - _ROW0
_PR, _PC = 40, 256
_NWIN = 4
_WROW = (32, 80, 120, 168)
_WCOL = (0, 128, 128, 256)


def _const_maps():
    ks = np.arange(_K)
    x1 = 8 + 56 * ks
    y1 = 16 + 20 * ks
    w = 24 + 2 * ks
    h = 48 + 4 * ks
    x2, y2 = x1 + w, y1 + h
    cx = (x1 + x2) // 2
    cy = (y1 + y2) // 2

    gauss = np.zeros((_H, _W), np.float32)
    pos = np.zeros((_H, _W), np.float32)
    for k in range(_K):
        R = float(np.sqrt(float(cx[k]) ** 2 + float(cy[k]) ** 2))
        xm = np.tile(np.arange(w[k]), (h[k], 1)).astype(np.float32)
        ym = np.tile(np.arange(h[k]), (w[k], 1)).T.astype(np.float32)
        G = np.sqrt((xm - float(cx[k])) ** 2 + (ym - float(cy[k])) ** 2)
        kG = np.exp(-0.5 * G / R).astype(np.float32)
        cur = gauss[y1[k]:y2[k], x1[k]:x2[k]]
        gauss[y1[k]:y2[k], x1[k]:x2[k]] = np.maximum(kG, cur)
        pos[y1[k]:y2[k], x1[k]:x2[k]] = 1.0

    V = (np.power(1.0 - gauss, _BETA) * pos).astype(np.float32)
    V[cy, cx] = -1.0

    logh = np.log(h.astype(np.float32))
    tp = np.zeros((_NWIN * _PR, _PC), np.float32)
    for k in range(_K):
        j = k // 2
        for d in (-2, -1, 0, 1, 2):
            tp[j * _PR + cy[k] + d - _WROW[j], cx[k] + d - _WCOL[j]] = logh[k]
    return V[_ROW0:_ROW1], tp


_V_MAP, _TP_MAP = _const_maps()


def _body(cm_any, sm_any, v_any, tp_any, c_ref, s_ref, cm_v, v_v, tp_v, scr,
          cm_sem, v_sem, tp_sem, sem):
    pltpu.make_async_copy(
        cm_any.at[0, 0, pl.ds(_ROW0, _NROWS), :], cm_v, cm_sem,
    ).start()
    pltpu.make_async_copy(v_any, v_v, v_sem).start()
    pltpu.make_async_copy(tp_any, tp_v, tp_sem).start()
    for j in range(_NWIN):
        pltpu.make_async_copy(
            sm_any.at[0, 0, pl.ds(_WROW[j], _PR), pl.ds(_WCOL[j], _PC)],
            scr.at[pl.ds(j * _PR, _PR)], sem,
        ).start()

    pltpu.make_async_copy(
        cm_any.at[0, 0, pl.ds(_ROW0, _NROWS), :], cm_v, cm_sem,
    ).wait()
    pltpu.make_async_copy(v_any, v_v, v_sem).wait()
    p = jnp.clip(cm_v[...], 0.0001, 1.0 - 0.0001)
    v = v_v[...].astype(jnp.float32)
    q = jnp.where(v < 0.0, 1.0 - p, p)
    c_ref[0, 0] = jnp.sum(jnp.abs(v) * q * q * (-jnp.log(1.0 - q))) * (1.0 / _K)

    for j in range(_NWIN):
        pltpu.make_async_copy(
            sm_any.at[0, 0, pl.ds(_WROW[j], _PR), pl.ds(_WCOL[j], _PC)],
            scr.at[pl.ds(j * _PR, _PR)], sem,
        ).wait()
    pltpu.make_async_copy(tp_any, tp_v, tp_sem).wait()
    t = tp_v[...]
    d = jnp.abs(t - scr[...])
    sl = jnp.where(d <= 1.0, 0.5 * d * d, d - 0.5)
    s_ref[0, 0] = jnp.sum(jnp.where(t != 0.0, sl, 0.0)) * (1.0 / _K)


def kernel(center_maps, scale_maps, annotations, stride=4):
    c, s = pl.pallas_call(
        _body,
        in_specs=[
            pl.BlockSpec(memory_space=pl.ANY),
            pl.BlockSpec(memory_space=pl.ANY),
            pl.BlockSpec(memory_space=pl.ANY),
            pl.BlockSpec(memory_space=pl.ANY),
        ],
        out_specs=(
            pl.BlockSpec(memory_space=pltpu.SMEM),
            pl.BlockSpec(memory_space=pltpu.SMEM),
        ),
        out_shape=(
            jax.ShapeDtypeStruct((1, 1), jnp.float32),
            jax.ShapeDtypeStruct((1, 1), jnp.float32),
        ),
        scratch_shapes=[
            pltpu.VMEM((_NROWS, _W), jnp.float32),
            pltpu.VMEM((_NROWS, _W), jnp.bfloat16),
            pltpu.VMEM((_NWIN * _PR, _PC), jnp.float32),
            pltpu.VMEM((_NWIN * _PR, _PC), jnp.float32),
            pltpu.SemaphoreType.DMA,
            pltpu.SemaphoreType.DMA,
            pltpu.SemaphoreType.DMA,
            pltpu.SemaphoreType.DMA,
        ],
    )(center_maps, scale_maps, jnp.asarray(_V_MAP, dtype=jnp.bfloat16), jnp.asarray(_TP_MAP))
    return (c.reshape(1), s.reshape(1))


# speedup vs baseline: 1.0505x; 1.0162x over previous
import numpy as np
import jax
import jax.numpy as jnp
from jax.experimental import pallas as pl
from jax.experimental.pallas import tpu as pltpu

_ALPHA, _GAMMA, _BETA = 1.0, 2.0, 4.0
_B, _C, _H, _W = 16, 1, 256, 512
_K = 8
_ROW0, _ROW1 = 16, 232
_NROWS = _ROW1 - _ROW0
_PR, _PC = 16, 128


def _const_maps():
    ks = np.arange(_K)
    x1 = 8 + 56 * ks
    y1 = 16 + 20 * ks
    w = 24 + 2 * ks
    h = 48 + 4 * ks
    x2, y2 = x1 + w, y1 + h
    cx = (x1 + x2) // 2
    cy = (y1 + y2) // 2

    gauss = np.zeros((_H, _W), np.float32)
    pos = np.zeros((_H, _W), np.float32)
    for k in range(_K):
        R = float(np.sqrt(float(cx[k]) ** 2 + float(cy[k]) ** 2))
        xm = np.tile(np.arange(w[k]), (h[k], 1)).astype(np.float32)
        ym = np.tile(np.arange(h[k]), (w[k], 1)).T.astype(np.float32)
        G = np.sqrt((xm - float(cx[k])) ** 2 + (ym - float(cy[k])) ** 2)
        kG = np.exp(-0.5 * G / R).astype(np.float32)
        cur = gauss[y1[k]:y2[k], x1[k]:x2[k]]
        gauss[y1[k]:y2[k], x1[k]:x2[k]] = np.maximum(kG, cur)
        pos[y1[k]:y2[k], x1[k]:x2[k]] = 1.0

    V = (np.power(1.0 - gauss, _BETA) * pos).astype(np.float32)
    V[cy, cx] = -1.0

    logh = np.log(h.astype(np.float32))
    prow = ((cy - 2) // 8) * 8
    pcol = ((cx - 2) // _PC) * _PC
    tp = np.zeros((_K, _PR, _PC), np.float32)
    for k in range(_K):
        for d in (-2, -1, 0, 1, 2):
            tp[k, cy[k] + d - prow[k], cx[k] + d - pcol[k]] = logh[k]
    return V[_ROW0:_ROW1], tp, prow, pcol


_V_MAP, _TP_MAP, _PROW, _PCOL = _const_maps()


def _body(cm_any, sm_any, v_any, tp_any, c_ref, s_ref, cm_v, v_v, tp_v, scr,
          cm_sem, v_sem, tp_sem, sem):
    pltpu.make_async_copy(
        cm_any.at[0, 0, pl.ds(_ROW0, _NROWS), :], cm_v, cm_sem,
    ).start()
    pltpu.make_async_copy(v_any, v_v, v_sem).start()
    pltpu.make_async_copy(tp_any, tp_v, tp_sem).start()
    for k in range(_K):
        pltpu.make_async_copy(
            sm_any.at[0, 0, pl.ds(int(_PROW[k]), _PR),
                      pl.ds(int(_PCOL[k]), _PC)],
            scr.at[k], sem,
        ).start()

    pltpu.make_async_copy(
        cm_any.at[0, 0, pl.ds(_ROW0, _NROWS), :], cm_v, cm_sem,
    ).wait()
    pltpu.make_async_copy(v_any, v_v, v_sem).wait()
    p = jnp.clip(cm_v[...], 0.0001, 1.0 - 0.0001)
    v = v_v[...].astype(jnp.float32)
    q = jnp.where(v < 0.0, 1.0 - p, p)
    c_ref[0, 0] = jnp.sum(jnp.abs(v) * q * q * (-jnp.log(1.0 - q))) * (1.0 / _K)

    for k in range(_K):
        pltpu.make_async_copy(
            sm_any.at[0, 0, pl.ds(int(_PROW[k]), _PR),
                      pl.ds(int(_PCOL[k]), _PC)],
            scr.at[k], sem,
        ).wait()
    pltpu.make_async_copy(tp_any, tp_v, tp_sem).wait()
    t = tp_v[...]
    d = jnp.abs(t - scr[...])
    sl = jnp.where(d <= 1.0, 0.5 * d * d, d - 0.5)
    s_ref[0, 0] = jnp.sum(jnp.where(t != 0.0, sl, 0.0)) * (1.0 / _K)


def kernel(center_maps, scale_maps, annotations, stride=4):
    c, s = pl.pallas_call(
        _body,
        in_specs=[
            pl.BlockSpec(memory_space=pl.ANY),
            pl.BlockSpec(memory_space=pl.ANY),
            pl.BlockSpec(memory_space=pl.ANY),
            pl.BlockSpec(memory_space=pl.ANY),
        ],
        out_specs=(
            pl.BlockSpec(memory_space=pltpu.SMEM),
            pl.BlockSpec(memory_space=pltpu.SMEM),
        ),
        out_shape=(
            jax.ShapeDtypeStruct((1, 1), jnp.float32),
            jax.ShapeDtypeStruct((1, 1), jnp.float32),
        ),
        scratch_shapes=[
            pltpu.VMEM((_NROWS, _W), jnp.float32),
            pltpu.VMEM((_NROWS, _W), jnp.bfloat16),
            pltpu.VMEM((_K, _PR, _PC), jnp.float32),
            pltpu.VMEM((_K, _PR, _PC), jnp.float32),
            pltpu.SemaphoreType.DMA,
            pltpu.SemaphoreType.DMA,
            pltpu.SemaphoreType.DMA,
            pltpu.SemaphoreType.DMA,
        ],
    )(center_maps, scale_maps, jnp.asarray(_V_MAP, dtype=jnp.bfloat16), jnp.asarray(_TP_MAP))
    return (c.reshape(1), s.reshape(1))


# speedup vs baseline: 1.0553x; 1.0046x over previous
import numpy as np
import jax
import jax.numpy as jnp
from jax.experimental import pallas as pl
from jax.experimental.pallas import tpu as pltpu

_ALPHA, _GAMMA, _BETA = 1.0, 2.0, 4.0
_B, _C, _H, _W = 16, 1, 256, 512
_K = 8
_ROW0, _ROW1 = 16, 232
_NROWS = _ROW1 - _ROW0
_PR, _PC = 16, 128


def _const_maps():
    ks = np.arange(_K)
    x1 = 8 + 56 * ks
    y1 = 16 + 20 * ks
    w = 24 + 2 * ks
    h = 48 + 4 * ks
    x2, y2 = x1 + w, y1 + h
    cx = (x1 + x2) // 2
    cy = (y1 + y2) // 2

    gauss = np.zeros((_H, _W), np.float32)
    pos = np.zeros((_H, _W), np.float32)
    for k in range(_K):
        R = float(np.sqrt(float(cx[k]) ** 2 + float(cy[k]) ** 2))
        xm = np.tile(np.arange(w[k]), (h[k], 1)).astype(np.float32)
        ym = np.tile(np.arange(h[k]), (w[k], 1)).T.astype(np.float32)
        G = np.sqrt((xm - float(cx[k])) ** 2 + (ym - float(cy[k])) ** 2)
        kG = np.exp(-0.5 * G / R).astype(np.float32)
        cur = gauss[y1[k]:y2[k], x1[k]:x2[k]]
        gauss[y1[k]:y2[k], x1[k]:x2[k]] = np.maximum(kG, cur)
        pos[y1[k]:y2[k], x1[k]:x2[k]] = 1.0

    V = (np.power(1.0 - gauss, _BETA) * pos).astype(np.float32)
    V[cy, cx] = -1.0

    logh = np.log(h.astype(np.float32))
    prow = ((cy - 2) // 8) * 8
    pcol = ((cx - 2) // _PC) * _PC
    tp = np.zeros((_K, _PR, _PC), np.float32)
    for k in range(_K):
        for d in (-2, -1, 0, 1, 2):
            tp[k, cy[k] + d - prow[k], cx[k] + d - pcol[k]] = logh[k]
    return V[_ROW0:_ROW1], tp, prow, pcol


_V_MAP, _TP_MAP, _PROW, _PCOL = _const_maps()


def _body(cm_any, sm_any, v_any, tp_any, c_ref, s_ref, cm_v, v_v, tp_v, scr,
          cm_sem, v_sem, tp_sem, sem):
    pltpu.make_async_copy(
        cm_any.at[0, 0, pl.ds(_ROW0, _NROWS), :], cm_v, cm_sem,
    ).start()
    pltpu.make_async_copy(v_any, v_v, v_sem).start()
    pltpu.make_async_copy(tp_any, tp_v, tp_sem).start()
    for k in range(_K):
        pltpu.make_async_copy(
            sm_any.at[0, 0, pl.ds(int(_PROW[k]), _PR),
                      pl.ds(int(_PCOL[k]), _PC)],
            scr.at[k], sem,
        ).start()

    for k in range(_K):
        pltpu.make_async_copy(
            sm_any.at[0, 0, pl.ds(int(_PROW[k]), _PR),
                      pl.ds(int(_PCOL[k]), _PC)],
            scr.at[k], sem,
        ).wait()
    pltpu.make_async_copy(tp_any, tp_v, tp_sem).wait()
    t = tp_v[...]
    d = jnp.abs(t - scr[...])
    sl = jnp.where(d <= 1.0, 0.5 * d * d, d - 0.5)
    s_ref[0, 0] = jnp.sum(jnp.where(t != 0.0, sl, 0.0)) * (1.0 / _K)

    pltpu.make_async_copy(
        cm_any.at[0, 0, pl.ds(_ROW0, _NROWS), :], cm_v, cm_sem,
    ).wait()
    pltpu.make_async_copy(v_any, v_v, v_sem).wait()
    p = jnp.clip(cm_v[...], 0.0001, 1.0 - 0.0001)
    v = v_v[...].astype(jnp.float32)
    q = jnp.where(v < 0.0, 1.0 - p, p)
    c_ref[0, 0] = jnp.sum(jnp.abs(v) * q * q * (-jnp.log(1.0 - q))) * (1.0 / _K)


def kernel(center_maps, scale_maps, annotations, stride=4):
    c, s = pl.pallas_call(
        _body,
        in_specs=[
            pl.BlockSpec(memory_space=pl.ANY),
            pl.BlockSpec(memory_space=pl.ANY),
            pl.BlockSpec(memory_space=pl.ANY),
            pl.BlockSpec(memory_space=pl.ANY),
        ],
        out_specs=(
            pl.BlockSpec(memory_space=pltpu.SMEM),
            pl.BlockSpec(memory_space=pltpu.SMEM),
        ),
        out_shape=(
            jax.ShapeDtypeStruct((1, 1), jnp.float32),
            jax.ShapeDtypeStruct((1, 1), jnp.float32),
        ),
        scratch_shapes=[
            pltpu.VMEM((_NROWS, _W), jnp.float32),
            pltpu.VMEM((_NROWS, _W), jnp.bfloat16),
            pltpu.VMEM((_K, _PR, _PC), jnp.float32),
            pltpu.VMEM((_K, _PR, _PC), jnp.float32),
            pltpu.SemaphoreType.DMA,
            pltpu.SemaphoreType.DMA,
            pltpu.SemaphoreType.DMA,
            pltpu.SemaphoreType.DMA,
        ],
    )(center_maps, scale_maps, jnp.asarray(_V_MAP, dtype=jnp.bfloat16), jnp.asarray(_TP_MAP))
    return (c.reshape(1), s.reshape(1))
